# baseline (speedup 1.0000x reference)
import functools

import jax
import jax.numpy as jnp
from jax import lax
from jax.experimental import pallas as pl
from jax.experimental.pallas import tpu as pltpu
from jax.experimental.pallas import tpu_sc as plsc

_NC = 2
_NS = 16
_NW = _NC * _NS
_BBLK = 128
_NL = 4
_BPAD = 129


@functools.lru_cache(maxsize=None)
def _make_gather(b, l, vocab, dim):
    assert b == _NW * _BBLK and dim == 32
    nchunk = l // _NL
    niter = nchunk // 2

    mesh = plsc.VectorSubcoreMesh(core_axis_name="c", subcore_axis_name="s")

    @functools.partial(
        pl.kernel,
        mesh=mesh,
        out_type=jax.ShapeDtypeStruct((l, dim, b), jnp.float32),
        scratch_types=[
            pltpu.VMEM((l, _BBLK), jnp.int32),
            pltpu.VMEM((_NL * _BBLK, dim), jnp.float32),
            pltpu.VMEM((_NL * _BBLK, dim), jnp.float32),
            pltpu.VMEM((_NL * _BBLK, dim), jnp.float32),
            pltpu.VMEM((_NL * _BBLK, dim), jnp.float32),
            pltpu.VMEM((_NL, dim, _BPAD), jnp.float32),
            pltpu.VMEM((_NL, dim, _BPAD), jnp.float32),
            pltpu.SemaphoreType.DMA,
            pltpu.SemaphoreType.DMA,
            pltpu.SemaphoreType.DMA,
            pltpu.SemaphoreType.DMA,
            pltpu.SemaphoreType.DMA,
            pltpu.SemaphoreType.DMA,
        ],
        compiler_params=pltpu.CompilerParams(
            use_tc_tiling_on_sc=False, needs_layout_passes=False),
    )
    def gather_kernel(table_hbm, xt_hbm, out_hbm, idx_v, stg0, stg1, stg2,
                      stg3, blk0, blk1, gsem0, gsem1, gsem2, gsem3,
                      wsem0, wsem1):
        wid = lax.axis_index("s") * _NC + lax.axis_index("c")
        b0 = wid * _BBLK
        pltpu.sync_copy(xt_hbm.at[:, pl.ds(b0, _BBLK)], idx_v)

        lane_lo = jnp.arange(16, dtype=jnp.int32)
        lane_hi = lane_lo + 16

        def fire(g, stg, sem):
            l0 = g * _NL
            return [
                pltpu.async_copy(
                    table_hbm.at[idx_v.at[l0 + j]],
                    stg.at[pl.ds(j * _BBLK, _BBLK)],
                    sem,
                )
                for j in range(_NL)
            ]

        _UNROLL = 16

        def transpose_li(stg, blk, li):
            li_ids = jnp.full((16,), li, dtype=jnp.int32)
            row0 = li * _BBLK

            def grp_body(g8, carry):
                bi0 = g8 * _UNROLL
                for k in range(_UNROLL):
                    bi = bi0 + k
                    r = row0 + bi
                    b_ids = jnp.full((16,), bi, dtype=jnp.int32)
                    plsc.store_scatter(
                        blk, [li_ids, lane_lo, b_ids], stg[r, pl.ds(0, 16)])
                    plsc.store_scatter(
                        blk, [li_ids, lane_hi, b_ids], stg[r, pl.ds(16, 16)])
                return carry

            lax.fori_loop(0, _BBLK // _UNROLL, grp_body, 0)

        def drain(stg, gsem):
            pltpu.make_async_copy(
                table_hbm.at[pl.ds(0, _NL * _BBLK)], stg, gsem).wait()

        def reclaim(blk, wsem):
            pltpu.make_async_copy(
                blk.at[:, :, pl.ds(0, _BBLK)],
                out_hbm.at[pl.ds(0, _NL), :, pl.ds(b0, _BBLK)], wsem).wait()

        def handle(g, stg, blk, gsem, wsem, first, refill):
            drain(stg, gsem)

            if first is None:
                reclaim(blk, wsem)
            else:
                @pl.when(jnp.logical_not(first))
                def _():
                    reclaim(blk, wsem)

            for li in range(_NL):
                transpose_li(stg, blk, li)

            if refill:
                @pl.when(g + 4 < nchunk)
                def _():
                    fire(g + 4, stg, gsem)

            pltpu.async_copy(
                blk.at[:, :, pl.ds(0, _BBLK)],
                out_hbm.at[pl.ds(g * _NL, _NL), :, pl.ds(b0, _BBLK)], wsem)

        fire(0, stg0, gsem0)
        fire(1, stg1, gsem1)
        fire(2, stg2, gsem2)
        fire(3, stg3, gsem3)

        def body(j, carry):
            g = 4 * j
            handle(g, stg0, blk0, gsem0, wsem0, j == 0, True)
            handle(g + 1, stg1, blk1, gsem1, wsem1, j == 0, True)
            handle(g + 2, stg2, blk0, gsem2, wsem0, None, True)
            handle(g + 3, stg3, blk1, gsem3, wsem1, None, True)
            return carry

        lax.fori_loop(0, nchunk // 4, body, 0)
        handle(nchunk - 2, stg0, blk0, gsem0, wsem0, None, False)
        handle(nchunk - 1, stg1, blk1, gsem1, wsem1, None, False)
        pltpu.make_async_copy(
            blk0.at[:, :, pl.ds(0, _BBLK)], out_hbm.at[pl.ds(0, _NL), :, pl.ds(b0, _BBLK)], wsem0).wait()
        pltpu.make_async_copy(
            blk1.at[:, :, pl.ds(0, _BBLK)], out_hbm.at[pl.ds(0, _NL), :, pl.ds(b0, _BBLK)], wsem1).wait()

    return gather_kernel


def kernel(x, table):
    b, l = x.shape
    vocab, dim = table.shape
    xt = x.T.astype(jnp.int32)
    out_phys = _make_gather(b, l, vocab, dim)(table, xt)
    return out_phys.transpose(2, 0, 1)


# speedup vs baseline: 1.0115x; 1.0115x over previous
import functools

import jax
import jax.numpy as jnp
from jax import lax
from jax.experimental import pallas as pl
from jax.experimental.pallas import tpu as pltpu
from jax.experimental.pallas import tpu_sc as plsc

_NC = 2
_NS = 16
_NW = _NC * _NS
_BBLK = 128
_NL = 4
_BPAD = 129


@functools.lru_cache(maxsize=None)
def _make_gather(b, l, vocab, dim):
    assert b == _NW * _BBLK and dim == 32
    nchunk = l // _NL
    niter = nchunk // 2

    mesh = plsc.VectorSubcoreMesh(core_axis_name="c", subcore_axis_name="s")

    @functools.partial(
        pl.kernel,
        mesh=mesh,
        out_type=jax.ShapeDtypeStruct((l, dim, b), jnp.float32),
        scratch_types=[
            pltpu.VMEM((l, _BBLK), jnp.int32),
            pltpu.VMEM((_NL * _BBLK, dim), jnp.float32),
            pltpu.VMEM((_NL * _BBLK, dim), jnp.float32),
            pltpu.VMEM((_NL * _BBLK, dim), jnp.float32),
            pltpu.VMEM((_NL * _BBLK, dim), jnp.float32),
            pltpu.VMEM((_NL, dim, _BPAD), jnp.float32),
            pltpu.VMEM((_NL, dim, _BPAD), jnp.float32),
            pltpu.SemaphoreType.DMA,
            pltpu.SemaphoreType.DMA,
            pltpu.SemaphoreType.DMA,
            pltpu.SemaphoreType.DMA,
            pltpu.SemaphoreType.DMA,
            pltpu.SemaphoreType.DMA,
        ],
        compiler_params=pltpu.CompilerParams(
            use_tc_tiling_on_sc=False, needs_layout_passes=False),
    )
    def gather_kernel(table_hbm, xt_hbm, out_hbm, idx_v, stg0, stg1, stg2,
                      stg3, blk0, blk1, gsem0, gsem1, gsem2, gsem3,
                      wsem0, wsem1):
        wid = lax.axis_index("s") * _NC + lax.axis_index("c")
        b0 = wid * _BBLK
        pltpu.sync_copy(xt_hbm.at[:, pl.ds(b0, _BBLK)], idx_v)

        lane_lo = jnp.arange(16, dtype=jnp.int32)
        lane_hi = lane_lo + 16

        def fire(g, stg, sem):
            l0 = g * _NL
            return [
                pltpu.async_copy(
                    table_hbm.at[idx_v.at[l0 + j]],
                    stg.at[pl.ds(j * _BBLK, _BBLK)],
                    sem,
                )
                for j in range(_NL)
            ]

        _UNROLL = 4

        def transpose_li(stg, blk, li):
            li_ids = jnp.full((16,), li, dtype=jnp.int32)
            row0 = li * _BBLK

            def grp_body(g8, carry):
                bi0 = g8 * _UNROLL
                for k in range(_UNROLL):
                    bi = bi0 + k
                    r = row0 + bi
                    b_ids = jnp.full((16,), bi, dtype=jnp.int32)
                    plsc.store_scatter(
                        blk, [li_ids, lane_lo, b_ids], stg[r, pl.ds(0, 16)])
                    plsc.store_scatter(
                        blk, [li_ids, lane_hi, b_ids], stg[r, pl.ds(16, 16)])
                return carry

            lax.fori_loop(0, _BBLK // _UNROLL, grp_body, 0)

        def drain(stg, gsem):
            pltpu.make_async_copy(
                table_hbm.at[pl.ds(0, _NL * _BBLK)], stg, gsem).wait()

        def reclaim(blk, wsem):
            pltpu.make_async_copy(
                blk.at[:, :, pl.ds(0, _BBLK)],
                out_hbm.at[pl.ds(0, _NL), :, pl.ds(b0, _BBLK)], wsem).wait()

        def handle(g, stg, blk, gsem, wsem, first, refill):
            drain(stg, gsem)

            if first is None:
                reclaim(blk, wsem)
            else:
                @pl.when(jnp.logical_not(first))
                def _():
                    reclaim(blk, wsem)

            for li in range(_NL):
                transpose_li(stg, blk, li)

            if refill:
                @pl.when(g + 4 < nchunk)
                def _():
                    fire(g + 4, stg, gsem)

            pltpu.async_copy(
                blk.at[:, :, pl.ds(0, _BBLK)],
                out_hbm.at[pl.ds(g * _NL, _NL), :, pl.ds(b0, _BBLK)], wsem)

        fire(0, stg0, gsem0)
        fire(1, stg1, gsem1)
        fire(2, stg2, gsem2)
        fire(3, stg3, gsem3)

        def body(j, carry):
            g = 4 * j
            handle(g, stg0, blk0, gsem0, wsem0, j == 0, True)
            handle(g + 1, stg1, blk1, gsem1, wsem1, j == 0, True)
            handle(g + 2, stg2, blk0, gsem2, wsem0, None, True)
            handle(g + 3, stg3, blk1, gsem3, wsem1, None, True)
            return carry

        lax.fori_loop(0, nchunk // 4, body, 0)
        handle(nchunk - 2, stg0, blk0, gsem0, wsem0, None, False)
        handle(nchunk - 1, stg1, blk1, gsem1, wsem1, None, False)
        pltpu.make_async_copy(
            blk0.at[:, :, pl.ds(0, _BBLK)], out_hbm.at[pl.ds(0, _NL), :, pl.ds(b0, _BBLK)], wsem0).wait()
        pltpu.make_async_copy(
            blk1.at[:, :, pl.ds(0, _BBLK)], out_hbm.at[pl.ds(0, _NL), :, pl.ds(b0, _BBLK)], wsem1).wait()

    return gather_kernel


def kernel(x, table):
    b, l = x.shape
    vocab, dim = table.shape
    xt = x.T.astype(jnp.int32)
    out_phys = _make_gather(b, l, vocab, dim)(table, xt)
    return out_phys.transpose(2, 0, 1)


# speedup vs baseline: 1.0147x; 1.0031x over previous
import functools

import jax
import jax.numpy as jnp
from jax import lax
from jax.experimental import pallas as pl
from jax.experimental.pallas import tpu as pltpu
from jax.experimental.pallas import tpu_sc as plsc

_NC = 2
_NS = 16
_NW = _NC * _NS
_BBLK = 128
_NL = 4
_BPAD = 129


@functools.lru_cache(maxsize=None)
def _make_gather(b, l, vocab, dim):
    assert b == _NW * _BBLK and dim == 32
    nchunk = l // _NL
    niter = nchunk // 2

    mesh = plsc.VectorSubcoreMesh(core_axis_name="c", subcore_axis_name="s")

    @functools.partial(
        pl.kernel,
        mesh=mesh,
        out_type=jax.ShapeDtypeStruct((l, dim, b), jnp.float32),
        scratch_types=[
            pltpu.VMEM((l, _BBLK), jnp.int32),
            pltpu.VMEM((_NL * _BBLK, dim), jnp.float32),
            pltpu.VMEM((_NL * _BBLK, dim), jnp.float32),
            pltpu.VMEM((_NL * _BBLK, dim), jnp.float32),
            pltpu.VMEM((_NL * _BBLK, dim), jnp.float32),
            pltpu.VMEM((_NL, dim, _BPAD), jnp.float32),
            pltpu.VMEM((_NL, dim, _BPAD), jnp.float32),
            pltpu.SemaphoreType.DMA,
            pltpu.SemaphoreType.DMA,
            pltpu.SemaphoreType.DMA,
            pltpu.SemaphoreType.DMA,
            pltpu.SemaphoreType.DMA,
            pltpu.SemaphoreType.DMA,
        ],
        compiler_params=pltpu.CompilerParams(
            use_tc_tiling_on_sc=False, needs_layout_passes=False),
    )
    def gather_kernel(table_hbm, xt_hbm, out_hbm, idx_v, stg0, stg1, stg2,
                      stg3, blk0, blk1, gsem0, gsem1, gsem2, gsem3,
                      wsem0, wsem1):
        wid = lax.axis_index("s") * _NC + lax.axis_index("c")
        b0 = wid * _BBLK
        pltpu.sync_copy(xt_hbm.at[:, pl.ds(b0, _BBLK)], idx_v)

        lane_lo = jnp.arange(16, dtype=jnp.int32)
        lane_hi = lane_lo + 16

        def fire(g, stg, sem):
            l0 = g * _NL
            return [
                pltpu.async_copy(
                    table_hbm.at[idx_v.at[l0 + j]],
                    stg.at[pl.ds(j * _BBLK, _BBLK)],
                    sem,
                )
                for j in range(_NL)
            ]

        _UNROLL = 8

        def transpose_li(stg, blk, li):
            li_ids = jnp.full((16,), li, dtype=jnp.int32)
            row0 = li * _BBLK

            def grp_body(g8, carry):
                bi0 = g8 * _UNROLL
                for k in range(_UNROLL):
                    bi = bi0 + k
                    r = row0 + bi
                    b_ids = jnp.full((16,), bi, dtype=jnp.int32)
                    plsc.store_scatter(
                        blk, [li_ids, lane_lo, b_ids], stg[r, pl.ds(0, 16)])
                    plsc.store_scatter(
                        blk, [li_ids, lane_hi, b_ids], stg[r, pl.ds(16, 16)])
                return carry

            lax.fori_loop(0, _BBLK // _UNROLL, grp_body, 0)

        def drain(stg, gsem):
            pltpu.make_async_copy(
                table_hbm.at[pl.ds(0, _NL * _BBLK)], stg, gsem).wait()

        def reclaim(blk, wsem):
            pltpu.make_async_copy(
                blk.at[:, :, pl.ds(0, _BBLK)],
                out_hbm.at[pl.ds(0, _NL), :, pl.ds(b0, _BBLK)], wsem).wait()

        def handle(g, stg, blk, gsem, wsem, first, refill):
            drain(stg, gsem)

            if first is None:
                reclaim(blk, wsem)
            else:
                @pl.when(jnp.logical_not(first))
                def _():
                    reclaim(blk, wsem)

            for li in range(_NL):
                transpose_li(stg, blk, li)

            if refill:
                @pl.when(g + 4 < nchunk)
                def _():
                    fire(g + 4, stg, gsem)

            pltpu.async_copy(
                blk.at[:, :, pl.ds(0, _BBLK)],
                out_hbm.at[pl.ds(g * _NL, _NL), :, pl.ds(b0, _BBLK)], wsem)

        fire(0, stg0, gsem0)
        fire(1, stg1, gsem1)
        fire(2, stg2, gsem2)
        fire(3, stg3, gsem3)

        def body(j, carry):
            g = 4 * j
            handle(g, stg0, blk0, gsem0, wsem0, j == 0, True)
            handle(g + 1, stg1, blk1, gsem1, wsem1, j == 0, True)
            handle(g + 2, stg2, blk0, gsem2, wsem0, None, True)
            handle(g + 3, stg3, blk1, gsem3, wsem1, None, True)
            return carry

        lax.fori_loop(0, nchunk // 4, body, 0)
        handle(nchunk - 2, stg0, blk0, gsem0, wsem0, None, False)
        handle(nchunk - 1, stg1, blk1, gsem1, wsem1, None, False)
        pltpu.make_async_copy(
            blk0.at[:, :, pl.ds(0, _BBLK)], out_hbm.at[pl.ds(0, _NL), :, pl.ds(b0, _BBLK)], wsem0).wait()
        pltpu.make_async_copy(
            blk1.at[:, :, pl.ds(0, _BBLK)], out_hbm.at[pl.ds(0, _NL), :, pl.ds(b0, _BBLK)], wsem1).wait()

    return gather_kernel


def kernel(x, table):
    b, l = x.shape
    vocab, dim = table.shape
    xt = x.T.astype(jnp.int32)
    out_phys = _make_gather(b, l, vocab, dim)(table, xt)
    return out_phys.transpose(2, 0, 1)
